# baseline (speedup 1.0000x reference)
import functools

import jax
import jax.numpy as jnp
from jax import lax
from jax.experimental import pallas as pl

NUM_RESAMPLE = 64
import numpy as np

TINY = np.float32(1.1754944e-38)


def _rotl(x, d):
    return lax.shift_left(x, jnp.uint32(d)) | lax.shift_right_logical(
        x, jnp.uint32(32 - d))


def _threefry_bits(cnt):
    k1 = jnp.uint32(0)
    k2 = jnp.uint32(42)
    k3 = jnp.uint32(0 ^ 42 ^ 0x1BD11BDA)
    ks = (k1, k2, k3)
    rot = ((13, 15, 26, 6), (17, 29, 16, 24))
    x0 = jnp.zeros_like(cnt)
    x1 = cnt + k2
    for d in range(5):
        for r in rot[d % 2]:
            x0 = x0 + x1
            x1 = _rotl(x1, r)
            x1 = x1 ^ x0
        x0 = x0 + ks[(d + 1) % 3]
        x1 = x1 + ks[(d + 2) % 3] + jnp.uint32(d + 1)
    return x0 ^ x1


def _sample_body(w_ref, inds_ref, neww_ref, *, br, s_dim):
    blk = pl.program_id(0)
    j_iota = lax.broadcasted_iota(jnp.uint32, (NUM_RESAMPLE, s_dim), 0)
    s_iota = lax.broadcasted_iota(jnp.uint32, (NUM_RESAMPLE, s_dim), 1)
    off = s_iota * jnp.uint32(NUM_RESAMPLE) + j_iota
    lane_i = lax.broadcasted_iota(jnp.int32, (NUM_RESAMPLE, s_dim), 1)
    per_ray = jnp.uint32(s_dim * NUM_RESAMPLE)
    for rloc in range(br):
        r = (blk * br + rloc).astype(jnp.uint32)
        w = w_ref[rloc, :].reshape(1, s_dim)
        wm = jnp.maximum(w, TINY)
        invw = jnp.float32(1.0) / wm
        norm = jnp.sum(wm)
        cnt = r * per_ray + off
        bits = _threefry_bits(cnt)
        fb = lax.shift_right_logical(bits, jnp.uint32(9)) | jnp.uint32(
            0x3F800000)
        u = jnp.maximum(lax.bitcast_convert_type(fb, jnp.float32)
                        - jnp.float32(1.0), TINY)
        val = -jnp.log(u) * invw
        m = jnp.min(val, axis=1, keepdims=True)
        hit = val == m
        idx = jnp.min(jnp.where(hit, lane_i, s_dim), axis=1,
                      keepdims=True)
        one = lane_i == idx
        wg = jnp.sum(jnp.where(one, w, jnp.float32(0.0)), axis=1,
                     keepdims=True)
        prob = jnp.maximum(wg, TINY) / norm
        neww = wg / (jnp.float32(NUM_RESAMPLE) * prob + jnp.float32(1e-8))
        inds_ref[0, :, rloc:rloc + 1] = idx
        neww_ref[0, :, rloc:rloc + 1] = neww


@functools.partial(jax.jit, static_argnames=("interpret",))
def _sample(weights, interpret=False):
    n_rays, s_dim = weights.shape
    br = 8
    body = functools.partial(_sample_body, br=br, s_dim=s_dim)
    inds_t, neww_t = pl.pallas_call(
        body,
        grid=(n_rays // br,),
        in_specs=[pl.BlockSpec((br, s_dim), lambda i: (i, 0))],
        out_specs=[
            pl.BlockSpec((1, NUM_RESAMPLE, br), lambda i: (i, 0, 0)),
            pl.BlockSpec((1, NUM_RESAMPLE, br), lambda i: (i, 0, 0)),
        ],
        out_shape=[
            jax.ShapeDtypeStruct((n_rays // br, NUM_RESAMPLE, br), jnp.int32),
            jax.ShapeDtypeStruct((n_rays // br, NUM_RESAMPLE, br),
                                 jnp.float32),
        ],
        interpret=interpret,
    )(weights)
    inds = inds_t.transpose(0, 2, 1).reshape(n_rays, NUM_RESAMPLE)
    neww = neww_t.transpose(0, 2, 1).reshape(n_rays, NUM_RESAMPLE)
    return inds, neww


def kernel(weights, points):
    inds, neww = _sample(weights)
    pts = jnp.take_along_axis(points, inds[..., None], axis=-2)
    return jnp.concatenate([pts, neww[..., None]], axis=-1)
